# baseline (speedup 1.0000x reference)
import functools
import jax
import jax.numpy as jnp
from jax import lax
from jax.experimental import pallas as pl
from jax.experimental.pallas import tpu as pltpu
from jax.experimental.pallas import tpu_sc as plsc

NC = 2
NS = 16
NW = NC * NS
LANE = 16


def _deg_kernel(n_nodes: int, n_rows: int, n_pad: int):
  rw = n_rows // NW

  mesh = plsc.VectorSubcoreMesh(
      core_axis_name="c", subcore_axis_name="s", num_cores=NC, num_subcores=NS)

  @functools.partial(
      pl.kernel,
      out_type=jax.ShapeDtypeStruct((NC * n_pad,), jnp.float32),
      mesh=mesh,
      compiler_params=pltpu.CompilerParams(needs_layout_passes=False),
      scratch_types=[
          pltpu.VMEM((rw, 128), jnp.int32),
          pltpu.VMEM((rw, 128), jnp.float32),
          pltpu.VMEM((640,), jnp.float32),
          pltpu.VMEM_SHARED((n_pad,), jnp.float32),
          pltpu.SemaphoreType.DMA,
      ],
  )
  def deg_k(dst_hbm, w_hbm, out_hbm, dst_v, w_v, zb_v, deg_sh, sem):
    cid = lax.axis_index("c")
    sid = lax.axis_index("s")
    wid = sid * NC + cid

    for i in range(640 // LANE):
      zb_v[pl.ds(i * LANE, LANE)] = jnp.zeros((LANE,), jnp.float32)

    @pl.when(sid == 0)
    def _zero():
      def zloop(j, c):
        pltpu.sync_copy(zb_v.at[pl.ds(0, 640)], deg_sh.at[pl.ds(j * 640, 640)])
        return c
      nfull = n_pad // 640
      lax.fori_loop(0, nfull, zloop, 0)
      rem = n_pad - nfull * 640
      if rem:
        pltpu.sync_copy(zb_v.at[pl.ds(0, rem)], deg_sh.at[pl.ds(nfull * 640, rem)])

    plsc.subcore_barrier()

    pltpu.sync_copy(dst_hbm.at[pl.ds(wid * rw, rw)], dst_v)
    pltpu.sync_copy(w_hbm.at[pl.ds(wid * rw, rw)], w_v)
    k = 8
    for g in range(rw // k):
      ds = []
      for j in range(k):
        r = g * k + j
        ds.append(pltpu.async_copy(
            w_v.at[r], deg_sh.at[dst_v.at[r]], sem, add=True))
      for d in ds:
        d.wait()

    plsc.subcore_barrier()

    @pl.when(sid == 0)
    def _drain():
      pltpu.sync_copy(deg_sh, out_hbm.at[pl.ds(cid * n_pad, n_pad)])

  return deg_k


def _tcfuse_kernel(t_steps: int, n_nodes: int, c_dim: int, nb: int):
  bn = n_nodes // nb
  ch = c_dim // 2

  def body(xm1, x0, xp1, cw3, gw, cb, degp, ybf_ref, yf_ref, dinv_ref):
    g0 = jnp.dot(cw3[0], gw[...], preferred_element_type=jnp.float32)
    g1 = jnp.dot(cw3[1], gw[...], preferred_element_type=jnp.float32)
    g2 = jnp.dot(cw3[2], gw[...], preferred_element_type=jnp.float32)
    acc = jnp.dot(xm1[0], g0, preferred_element_type=jnp.float32)
    acc = acc + jnp.dot(x0[0], g1, preferred_element_type=jnp.float32)
    acc = acc + jnp.dot(xp1[0], g2, preferred_element_type=jnp.float32)
    acc = acc + jnp.dot(cb[...], gw[...], preferred_element_type=jnp.float32)
    deg = degp[0] + degp[1] + 1.0
    dinv = lax.rsqrt(deg)
    y = acc * dinv
    for h in range(2):
      yh = y[:, h * ch:(h + 1) * ch]
      perm = yh.reshape(bn, 2, 2, 16).swapaxes(2, 3).reshape(bn, ch)
      ybf_ref[h, 0] = perm.astype(jnp.bfloat16)
    yf_ref[0] = y
    dinv_ref[...] = dinv

  return pl.pallas_call(
      body,
      grid=(t_steps, nb),
      in_specs=[
          pl.BlockSpec((1, bn, c_dim), lambda t, n: (t, n, 0)),
          pl.BlockSpec((1, bn, c_dim), lambda t, n: (t + 1, n, 0)),
          pl.BlockSpec((1, bn, c_dim), lambda t, n: (t + 2, n, 0)),
          pl.BlockSpec((3, c_dim, c_dim), lambda t, n: (0, 0, 0)),
          pl.BlockSpec((c_dim, c_dim), lambda t, n: (0, 0)),
          pl.BlockSpec((1, c_dim), lambda t, n: (0, 0)),
          pl.BlockSpec((NC, bn, 1), lambda t, n: (0, n, 0)),
      ],
      out_specs=[
          pl.BlockSpec((2, 1, bn, ch), lambda t, n: (0, t, n, 0)),
          pl.BlockSpec((1, bn, c_dim), lambda t, n: (t, n, 0)),
          pl.BlockSpec((bn, 1), lambda t, n: (n, 0)),
      ],
      out_shape=[
          jax.ShapeDtypeStruct((2, t_steps, n_nodes, ch), jnp.bfloat16),
          jax.ShapeDtypeStruct((t_steps, n_nodes, c_dim), jnp.float32),
          jax.ShapeDtypeStruct((n_nodes, 1), jnp.float32),
      ],
  )


def _agg_kernel(t_steps: int, n_nodes: int, c_dim: int, n_rows: int,
                n_pad: int):
  ch = c_dim // 2
  rt = n_rows // NS
  crows = rt // 2
  nchunks = 2
  stripe = n_pad // NS
  BE = 128
  NBUF = 4

  mesh = plsc.VectorSubcoreMesh(
      core_axis_name="c", subcore_axis_name="s", num_cores=NC, num_subcores=NS)

  @functools.partial(
      pl.kernel,
      out_type=[
          jax.ShapeDtypeStruct((t_steps * NC * n_pad, ch), jnp.float32),
          jax.ShapeDtypeStruct((NC * n_rows, 128), jnp.float32),
      ],
      mesh=mesh,
      compiler_params=pltpu.CompilerParams(
          needs_layout_passes=False, use_tc_tiling_on_sc=False),
      scratch_types=[
          pltpu.VMEM((n_pad,), jnp.float32),
          pltpu.VMEM((crows, 128), jnp.int32),
          pltpu.VMEM((crows, 128), jnp.int32),
          pltpu.VMEM((crows, 128), jnp.float32),
          pltpu.VMEM((128, 32), jnp.int32),
          pltpu.VMEM((128, 32), jnp.int32),
          pltpu.VMEM((128, 32), jnp.int32),
          pltpu.VMEM((128, 32), jnp.int32),
          pltpu.VMEM((128, 64), jnp.float32),
          pltpu.VMEM((128, 64), jnp.float32),
          pltpu.VMEM((128, 64), jnp.float32),
          pltpu.VMEM((128, 64), jnp.float32),
          pltpu.VMEM_SHARED((n_pad, 64), jnp.float32),
          pltpu.SemaphoreType.DMA,
          pltpu.SemaphoreType.DMA,
          pltpu.SemaphoreType.DMA,
          pltpu.SemaphoreType.DMA,
          pltpu.SemaphoreType.DMA,
          pltpu.SemaphoreType.DMA,
          pltpu.SemaphoreType.DMA,
          pltpu.SemaphoreType.DMA,
      ],
  )
  def agg_k(y_hbm, src_hbm, dst_hbm, w_hbm, dinv_hbm, part_hbm, z_hbm,
            dinv_v, src_v, dst_v, z_v, gb0, gb1, gb2, gb3,
            sb0, sb1, sb2, sb3, h_sh,
            sg0, sg1, sg2, sg3, ss0, ss1, ss2, ss3):
    cid = lax.axis_index("c")
    sid = lax.axis_index("s")
    gbufs = (gb0, gb1, gb2, gb3)
    sbufs = (sb0, sb1, sb2, sb3)
    semg = (sg0, sg1, sg2, sg3)
    sems = (ss0, ss1, ss2, ss3)

    pltpu.sync_copy(dinv_hbm, dinv_v)

    def zchunk(ck, c):
      row0 = sid * rt + ck * crows
      pltpu.sync_copy(dst_hbm.at[pl.ds(row0, crows)], dst_v)
      pltpu.sync_copy(w_hbm.at[pl.ds(row0, crows)], z_v)
      def zr(j, cc):
        for i in range(128 // LANE):
          sl = pl.ds(i * LANE, LANE)
          dv = plsc.load_gather(dinv_v, [dst_v[j, sl]])
          z_v[j, sl] = z_v[j, sl] * dv
        return cc
      lax.fori_loop(0, crows, zr, 0)
      pltpu.sync_copy(z_v, z_hbm.at[pl.ds(cid * n_rows + row0, crows)])
      return c
    lax.fori_loop(0, nchunks, zchunk, 0)

    zv = jnp.zeros((LANE,), jnp.float32)
    nb = crows
    tstride = t_steps * n_nodes

    def tloop(t, carry):
      def zrow(r, c):
        for j in range(ch // LANE):
          sb0[r, pl.ds(j * LANE, LANE)] = zv
        return c
      lax.fori_loop(0, 128, zrow, 0)
      base = sid * stripe
      nfull = stripe // 128
      for q in range(nfull):
        pltpu.sync_copy(sb0, h_sh.at[pl.ds(base + q * 128, 128)])
      rem = stripe - nfull * 128
      if rem:
        pltpu.sync_copy(sb0.at[pl.ds(0, rem)],
                        h_sh.at[pl.ds(base + nfull * 128, rem)])
      plsc.subcore_barrier()

      gbase = cid * tstride + t * n_nodes

      def chunk(ck, c):
        row0 = sid * rt + ck * crows
        pltpu.sync_copy(src_hbm.at[pl.ds(row0, crows)], src_v)
        pltpu.sync_copy(dst_hbm.at[pl.ds(row0, crows)], dst_v)
        pltpu.sync_copy(z_hbm.at[pl.ds(cid * n_rows + row0, crows)], z_v)
        def gix(j, cc):
          for i in range(128 // LANE):
            sl = pl.ds(i * LANE, LANE)
            src_v[j, sl] = src_v[j, sl] + gbase
          return cc
        lax.fori_loop(0, crows, gix, 0)

        pltpu.async_copy(y_hbm.at[src_v.at[0]], gbufs[0], semg[0])
        pltpu.async_copy(y_hbm.at[src_v.at[1]], gbufs[1], semg[1])

        def do_block(b, p):
          gbuf, sbuf, sg, ss = gbufs[p], sbufs[p], semg[p], sems[p]
          np_ = (p + 2) % NBUF

          @pl.when(b + 2 < nb)
          def _prefetch():
            @pl.when(b >= 2)
            def _wait_old_scatter():
              pltpu.make_async_copy(
                  sbufs[np_], h_sh.at[dst_v.at[b]], sems[np_]).wait()
            pltpu.async_copy(y_hbm.at[src_v.at[b + 2]], gbufs[np_], semg[np_])

          pltpu.make_async_copy(y_hbm.at[src_v.at[b]], gbuf, sg).wait()

          zrow_ref = z_v.at[b]
          @plsc.parallel_loop(0, 128, unroll=4)
          def scale(e):
            zb = plsc.load_gather(zrow_ref, [jnp.full((LANE,), e, jnp.int32)])
            for m in range(ch // 32):
              vw = gbuf[e, pl.ds(m * LANE, LANE)]
              vb = plsc.bitcast(vw, jnp.bfloat16)
              va, vc = plsc.unpack(vb, format=plsc.PackFormat.INTERLEAVED)
              sbuf[e, pl.ds(m * 32, LANE)] = va * zb
              sbuf[e, pl.ds(m * 32 + LANE, LANE)] = vc * zb

          pltpu.async_copy(sbuf, h_sh.at[dst_v.at[b]], ss, add=True)

        def blk(b, cc):
          for p in range(NBUF):
            @pl.when(b % NBUF == p)
            def _run(p=p):
              do_block(b, p)
          return cc
        lax.fori_loop(0, nb, blk, 0)

        for j in range(NBUF):
          bidx = nb - NBUF + j
          pltpu.make_async_copy(
              sbufs[bidx % NBUF], h_sh.at[dst_v.at[0]],
              sems[bidx % NBUF]).wait()
        return c
      lax.fori_loop(0, nchunks, chunk, 0)
      plsc.subcore_barrier()

      ob = (t * NC + cid) * n_pad + sid * stripe
      pltpu.sync_copy(h_sh.at[pl.ds(sid * stripe, stripe)],
                      part_hbm.at[pl.ds(ob, stripe)])
      return carry
    lax.fori_loop(0, t_steps, tloop, 0)

  return agg_k


def _bn_kernel(t_steps: int, n_nodes: int, c_dim: int, n_pad: int):
  ch = c_dim // 2

  def body(part, yf, dinv, gb, gamma, beta, out_ref):
    for h in range(2):
      lo, hi = h * ch, (h + 1) * ch
      hh = part[0, h, :n_nodes] + yf[0][:, lo:hi] * dinv[...] + gb[0:1, lo:hi]
      mu = jnp.mean(hh, axis=0, keepdims=True)
      var = jnp.mean((hh - mu) * (hh - mu), axis=0, keepdims=True)
      hn = ((hh - mu) * lax.rsqrt(var + 1e-5) * gamma[0:1, lo:hi]
            + beta[0:1, lo:hi])
      out_ref[0, :, lo:hi] = jnp.maximum(hn, 0.0)

  return pl.pallas_call(
      body,
      grid=(t_steps,),
      in_specs=[
          pl.BlockSpec((1, NC, n_pad, ch), lambda t: (t, 0, 0, 0)),
          pl.BlockSpec((1, n_nodes, c_dim), lambda t: (t, 0, 0)),
          pl.BlockSpec((n_nodes, 1), lambda t: (0, 0)),
          pl.BlockSpec((1, c_dim), lambda t: (0, 0)),
          pl.BlockSpec((1, c_dim), lambda t: (0, 0)),
          pl.BlockSpec((1, c_dim), lambda t: (0, 0)),
      ],
      out_specs=pl.BlockSpec((1, n_nodes, c_dim), lambda t: (t, 0, 0)),
      out_shape=jax.ShapeDtypeStruct((t_steps, n_nodes, c_dim), jnp.float32),
  )


def kernel(x, edge_index, edge_weight, conv_w, conv_b, gcn_w, gcn_b,
           bn_gamma, bn_beta):
  t_steps, n_nodes, c_dim = x.shape
  e_edges = edge_weight.shape[0]

  ce = 256
  per_w = -(-e_edges // (NW * ce)) * ce
  e2 = per_w * NW
  pad = e2 - e_edges
  fill = (jnp.arange(pad, dtype=jnp.int32) % n_nodes)
  src = jnp.concatenate([edge_index[0].astype(jnp.int32), fill])
  dst = jnp.concatenate([edge_index[1].astype(jnp.int32), fill])
  w = jnp.concatenate([edge_weight, jnp.zeros((pad,), jnp.float32)])
  n_rows = e2 // 128
  src2d = src.reshape(n_rows, 128)
  dst2d = dst.reshape(n_rows, 128)
  w2d = w.reshape(n_rows, 128)

  n_pad = -(-n_nodes // (NS * 8)) * (NS * 8)

  degp = _deg_kernel(n_nodes, n_rows, n_pad)(dst2d, w2d)
  degp = degp.reshape(NC, n_pad)[:, :n_nodes]

  xpad = jnp.concatenate(
      [jnp.zeros((1, n_nodes, c_dim), jnp.float32), x,
       jnp.zeros((1, n_nodes, c_dim), jnp.float32)], axis=0)
  cw3 = conv_w.transpose(2, 1, 0)
  ybf, yf, dinv2d = _tcfuse_kernel(t_steps, n_nodes, c_dim, 5)(
      xpad, xpad, xpad, cw3, gcn_w, conv_b.reshape(1, c_dim),
      degp.reshape(NC, n_nodes, 1))

  yflat = lax.bitcast_convert_type(
      ybf.reshape(NC, t_steps, n_nodes, c_dim // 4, 2), jnp.int32
  ).reshape(NC * t_steps * n_nodes, c_dim // 4)
  dinv1 = jnp.concatenate(
      [dinv2d.reshape(n_nodes), jnp.zeros((n_pad - n_nodes,), jnp.float32)])
  part, _ = _agg_kernel(t_steps, n_nodes, c_dim, n_rows, n_pad)(
      yflat, src2d, dst2d, w2d, dinv1)

  out = _bn_kernel(t_steps, n_nodes, c_dim, n_pad)(
      part.reshape(t_steps, NC, n_pad, c_dim // 2), yf, dinv2d,
      gcn_b.reshape(1, c_dim), bn_gamma.reshape(1, c_dim),
      bn_beta.reshape(1, c_dim))
  return out


# speedup vs baseline: 2.0096x; 2.0096x over previous
import functools
import jax
import jax.numpy as jnp
from jax import lax
from jax.experimental import pallas as pl
from jax.experimental.pallas import tpu as pltpu
from jax.experimental.pallas import tpu_sc as plsc

NC = 2
NS = 16
NW = NC * NS
LANE = 16


def _deg_kernel(n_nodes: int, n_rows: int, n_pad: int):
  rw = n_rows // NW

  mesh = plsc.VectorSubcoreMesh(
      core_axis_name="c", subcore_axis_name="s", num_cores=NC, num_subcores=NS)

  @functools.partial(
      pl.kernel,
      out_type=jax.ShapeDtypeStruct((NC * n_pad,), jnp.float32),
      mesh=mesh,
      compiler_params=pltpu.CompilerParams(needs_layout_passes=False),
      scratch_types=[
          pltpu.VMEM((rw, 128), jnp.int32),
          pltpu.VMEM((rw, 128), jnp.float32),
          pltpu.VMEM((640,), jnp.float32),
          pltpu.VMEM_SHARED((n_pad,), jnp.float32),
          pltpu.SemaphoreType.DMA,
      ],
  )
  def deg_k(dst_hbm, w_hbm, out_hbm, dst_v, w_v, zb_v, deg_sh, sem):
    cid = lax.axis_index("c")
    sid = lax.axis_index("s")
    wid = sid * NC + cid

    for i in range(640 // LANE):
      zb_v[pl.ds(i * LANE, LANE)] = jnp.zeros((LANE,), jnp.float32)

    @pl.when(sid == 0)
    def _zero():
      def zloop(j, c):
        pltpu.sync_copy(zb_v.at[pl.ds(0, 640)], deg_sh.at[pl.ds(j * 640, 640)])
        return c
      nfull = n_pad // 640
      lax.fori_loop(0, nfull, zloop, 0)
      rem = n_pad - nfull * 640
      if rem:
        pltpu.sync_copy(zb_v.at[pl.ds(0, rem)], deg_sh.at[pl.ds(nfull * 640, rem)])

    plsc.subcore_barrier()

    pltpu.sync_copy(dst_hbm.at[pl.ds(wid * rw, rw)], dst_v)
    pltpu.sync_copy(w_hbm.at[pl.ds(wid * rw, rw)], w_v)
    k = 8
    for g in range(rw // k):
      ds = []
      for j in range(k):
        r = g * k + j
        ds.append(pltpu.async_copy(
            w_v.at[r], deg_sh.at[dst_v.at[r]], sem, add=True))
      for d in ds:
        d.wait()

    plsc.subcore_barrier()

    @pl.when(sid == 0)
    def _drain():
      pltpu.sync_copy(deg_sh, out_hbm.at[pl.ds(cid * n_pad, n_pad)])

  return deg_k


def _tcfuse_kernel(t_steps: int, n_nodes: int, c_dim: int, nb: int):
  bn = n_nodes // nb

  def body(xm1, x0, xp1, cw3, gw, cb, degp, y_ref, dinv_ref):
    g0 = jnp.dot(cw3[0], gw[...], preferred_element_type=jnp.float32)
    g1 = jnp.dot(cw3[1], gw[...], preferred_element_type=jnp.float32)
    g2 = jnp.dot(cw3[2], gw[...], preferred_element_type=jnp.float32)
    acc = jnp.dot(xm1[0], g0, preferred_element_type=jnp.float32)
    acc = acc + jnp.dot(x0[0], g1, preferred_element_type=jnp.float32)
    acc = acc + jnp.dot(xp1[0], g2, preferred_element_type=jnp.float32)
    acc = acc + jnp.dot(cb[...], gw[...], preferred_element_type=jnp.float32)
    deg = degp[0] + degp[1] + 1.0
    dinv = lax.rsqrt(deg)
    y_ref[0] = acc * dinv
    dinv_ref[...] = dinv

  return pl.pallas_call(
      body,
      grid=(t_steps, nb),
      in_specs=[
          pl.BlockSpec((1, bn, c_dim), lambda t, n: (t, n, 0)),
          pl.BlockSpec((1, bn, c_dim), lambda t, n: (t + 1, n, 0)),
          pl.BlockSpec((1, bn, c_dim), lambda t, n: (t + 2, n, 0)),
          pl.BlockSpec((3, c_dim, c_dim), lambda t, n: (0, 0, 0)),
          pl.BlockSpec((c_dim, c_dim), lambda t, n: (0, 0)),
          pl.BlockSpec((1, c_dim), lambda t, n: (0, 0)),
          pl.BlockSpec((NC, bn, 1), lambda t, n: (0, n, 0)),
      ],
      out_specs=[
          pl.BlockSpec((1, bn, c_dim), lambda t, n: (t, n, 0)),
          pl.BlockSpec((bn, 1), lambda t, n: (n, 0)),
      ],
      out_shape=[
          jax.ShapeDtypeStruct((t_steps, n_nodes, c_dim), jnp.float32),
          jax.ShapeDtypeStruct((n_nodes, 1), jnp.float32),
      ],
  )


def _agg_kernel(t_steps: int, n_nodes: int, c_dim: int, n_rows: int,
                n_pad: int):
  rw = n_rows // NW
  crows = rw // 2
  nchunks = rw // crows
  stripe = n_pad // NS
  BE = 128
  drows = n_pad // 128

  mesh = plsc.VectorSubcoreMesh(
      core_axis_name="c", subcore_axis_name="s", num_cores=NC, num_subcores=NS)

  @functools.partial(
      pl.kernel,
      out_type=[
          jax.ShapeDtypeStruct((t_steps * NC * n_pad, c_dim), jnp.float32),
          jax.ShapeDtypeStruct((n_rows, 128), jnp.float32),
      ],
      mesh=mesh,
      compiler_params=pltpu.CompilerParams(needs_layout_passes=False),
      scratch_types=[
          pltpu.VMEM((crows, 128), jnp.int32),
          pltpu.VMEM((crows, 128), jnp.int32),
          pltpu.VMEM((crows, 128), jnp.float32),
          pltpu.VMEM((BE, c_dim), jnp.float32),
          pltpu.VMEM((BE, c_dim), jnp.float32),
          pltpu.VMEM_SHARED((n_pad, c_dim), jnp.float32),
          pltpu.SemaphoreType.DMA,
          pltpu.SemaphoreType.DMA,
          pltpu.SemaphoreType.DMA,
          pltpu.SemaphoreType.DMA,
      ],
  )
  def agg_k(y_hbm, src_hbm, dst_hbm, w_hbm, dinv_hbm, part_hbm, z_hbm,
            src_v, dst_v, z_v, rows_a, rows_b, h_sh,
            semg0, semg1, sems0, sems1):
    cid = lax.axis_index("c")
    sid = lax.axis_index("s")
    wid = sid * NC + cid
    bufs = (rows_a, rows_b)
    semg = (semg0, semg1)
    sems = (sems0, sems1)

    pltpu.sync_copy(dinv_hbm, rows_a.at[pl.ds(0, drows)])
    def zchunk(ck, c):
      row0 = wid * rw + ck * crows
      pltpu.sync_copy(dst_hbm.at[pl.ds(row0, crows)], dst_v)
      pltpu.sync_copy(w_hbm.at[pl.ds(row0, crows)], rows_b.at[pl.ds(0, crows)])
      def zr(j, cc):
        for i in range(128 // LANE):
          sl = pl.ds(i * LANE, LANE)
          d = dst_v[j, sl]
          dv = plsc.load_gather(
              rows_a, [lax.shift_right_logical(d, 7),
                       lax.bitwise_and(d, 127)])
          z_v[j, sl] = rows_b[j, sl] * dv
        return cc
      lax.fori_loop(0, crows, zr, 0)
      pltpu.sync_copy(z_v, z_hbm.at[pl.ds(row0, crows)])
      return c
    lax.fori_loop(0, nchunks, zchunk, 0)

    zv = jnp.zeros((LANE,), jnp.float32)
    nb = crows

    def tloop(t, carry):
      def zrow(r, c):
        for j in range(c_dim // LANE):
          rows_a[r, pl.ds(j * LANE, LANE)] = zv
        return c
      lax.fori_loop(0, BE, zrow, 0)
      base = sid * stripe
      nfull = stripe // BE
      for q in range(nfull):
        pltpu.sync_copy(rows_a, h_sh.at[pl.ds(base + q * BE, BE)])
      rem = stripe - nfull * BE
      if rem:
        pltpu.sync_copy(rows_a.at[pl.ds(0, rem)],
                        h_sh.at[pl.ds(base + nfull * BE, rem)])
      plsc.subcore_barrier()

      tN = t * n_nodes

      def chunk(ck, c):
        row0 = wid * rw + ck * crows
        pltpu.sync_copy(src_hbm.at[pl.ds(row0, crows)], src_v)
        pltpu.sync_copy(dst_hbm.at[pl.ds(row0, crows)], dst_v)
        pltpu.sync_copy(z_hbm.at[pl.ds(row0, crows)], z_v)
        def gix(j, cc):
          for i in range(128 // LANE):
            sl = pl.ds(i * LANE, LANE)
            src_v[j, sl] = src_v[j, sl] + tN
          return cc
        lax.fori_loop(0, crows, gix, 0)

        pltpu.async_copy(y_hbm.at[src_v.at[0]], bufs[0], semg[0])

        def do_block(b, p):
          buf, sg, ss = bufs[p], semg[p], sems[p]
          obuf, osg, oss = bufs[1 - p], semg[1 - p], sems[1 - p]

          @pl.when(b > 0)
          def _wait_other_scatter():
            pltpu.make_async_copy(obuf, h_sh.at[dst_v.at[b]], oss).wait()

          @pl.when(b + 1 < nb)
          def _prefetch():
            pltpu.async_copy(y_hbm.at[src_v.at[b + 1, pl.ds(0, 64)]],
                             obuf.at[pl.ds(0, 64)], osg)
            pltpu.async_copy(y_hbm.at[src_v.at[b + 1, pl.ds(64, 64)]],
                             obuf.at[pl.ds(64, 64)], osg)

          pltpu.make_async_copy(y_hbm.at[src_v.at[b]], buf, sg).wait()

          zrow_ref = z_v.at[b]
          @plsc.parallel_loop(0, BE, unroll=4)
          def scale(e):
            zb = plsc.load_gather(zrow_ref, [jnp.full((LANE,), e, jnp.int32)])
            for j in range(c_dim // LANE):
              sl = pl.ds(j * LANE, LANE)
              buf[e, sl] = buf[e, sl] * zb

          pltpu.async_copy(buf, h_sh.at[dst_v.at[b]], ss, add=True)

        def blk(b, cc):
          @pl.when(b % 2 == 0)
          def _even():
            do_block(b, 0)
          @pl.when(b % 2 == 1)
          def _odd():
            do_block(b, 1)
          return cc
        lax.fori_loop(0, nb, blk, 0)

        lp = (nb - 1) % 2
        pltpu.make_async_copy(
            bufs[lp], h_sh.at[dst_v.at[0]], sems[lp]).wait()
        return c
      lax.fori_loop(0, nchunks, chunk, 0)
      plsc.subcore_barrier()

      ob = (t * NC + cid) * n_pad + sid * stripe
      pltpu.sync_copy(h_sh.at[pl.ds(sid * stripe, stripe)],
                      part_hbm.at[pl.ds(ob, stripe)])
      return carry
    lax.fori_loop(0, t_steps, tloop, 0)

  return agg_k


def _bn_kernel(t_steps: int, n_nodes: int, c_dim: int, n_pad: int):
  def body(part, y, dinv, gb, gamma, beta, out_ref):
    h = (part[0, 0, :n_nodes] + part[0, 1, :n_nodes]
         + y[0] * dinv[...] + gb[...])
    mu = jnp.mean(h, axis=0, keepdims=True)
    var = jnp.mean((h - mu) * (h - mu), axis=0, keepdims=True)
    hn = (h - mu) * lax.rsqrt(var + 1e-5) * gamma[...] + beta[...]
    out_ref[0] = jnp.maximum(hn, 0.0)

  return pl.pallas_call(
      body,
      grid=(t_steps,),
      in_specs=[
          pl.BlockSpec((1, NC, n_pad, c_dim), lambda t: (t, 0, 0, 0)),
          pl.BlockSpec((1, n_nodes, c_dim), lambda t: (t, 0, 0)),
          pl.BlockSpec((n_nodes, 1), lambda t: (0, 0)),
          pl.BlockSpec((1, c_dim), lambda t: (0, 0)),
          pl.BlockSpec((1, c_dim), lambda t: (0, 0)),
          pl.BlockSpec((1, c_dim), lambda t: (0, 0)),
      ],
      out_specs=pl.BlockSpec((1, n_nodes, c_dim), lambda t: (t, 0, 0)),
      out_shape=jax.ShapeDtypeStruct((t_steps, n_nodes, c_dim), jnp.float32),
  )


def kernel(x, edge_index, edge_weight, conv_w, conv_b, gcn_w, gcn_b,
           bn_gamma, bn_beta):
  t_steps, n_nodes, c_dim = x.shape
  e_edges = edge_weight.shape[0]

  ce = 256
  per_w = -(-e_edges // (NW * ce)) * ce
  e2 = per_w * NW
  pad = e2 - e_edges
  fill = (jnp.arange(pad, dtype=jnp.int32) % n_nodes)
  src = jnp.concatenate([edge_index[0].astype(jnp.int32), fill])
  dst = jnp.concatenate([edge_index[1].astype(jnp.int32), fill])
  w = jnp.concatenate([edge_weight, jnp.zeros((pad,), jnp.float32)])
  n_rows = e2 // 128
  src2d = src.reshape(n_rows, 128)
  dst2d = dst.reshape(n_rows, 128)
  w2d = w.reshape(n_rows, 128)

  n_pad = -(-n_nodes // (NS * 8)) * (NS * 8)

  degp = _deg_kernel(n_nodes, n_rows, n_pad)(dst2d, w2d)
  degp = degp.reshape(NC, n_pad)[:, :n_nodes]

  xpad = jnp.concatenate(
      [jnp.zeros((1, n_nodes, c_dim), jnp.float32), x,
       jnp.zeros((1, n_nodes, c_dim), jnp.float32)], axis=0)
  cw3 = conv_w.transpose(2, 1, 0)
  y, dinv2d = _tcfuse_kernel(t_steps, n_nodes, c_dim, 5)(
      xpad, xpad, xpad, cw3, gcn_w, conv_b.reshape(1, c_dim),
      degp.reshape(NC, n_nodes, 1))

  yflat = y.reshape(t_steps * n_nodes, c_dim)
  dinv1 = jnp.concatenate(
      [dinv2d.reshape(n_nodes), jnp.zeros((n_pad - n_nodes,), jnp.float32)])
  part, _ = _agg_kernel(t_steps, n_nodes, c_dim, n_rows, n_pad)(
      yflat, src2d, dst2d, w2d, dinv1.reshape(n_pad // 128, 128))

  out = _bn_kernel(t_steps, n_nodes, c_dim, n_pad)(
      part.reshape(t_steps, NC, n_pad, c_dim), y, dinv2d,
      gcn_b.reshape(1, c_dim), bn_gamma.reshape(1, c_dim),
      bn_beta.reshape(1, c_dim))
  return out


# speedup vs baseline: 2.0106x; 1.0005x over previous
import functools
import jax
import jax.numpy as jnp
from jax import lax
from jax.experimental import pallas as pl
from jax.experimental.pallas import tpu as pltpu
from jax.experimental.pallas import tpu_sc as plsc

NC = 2
NS = 16
NW = NC * NS
LANE = 16


def _deg_kernel(n_nodes: int, n_rows: int, n_pad: int):
  rw = n_rows // NW

  mesh = plsc.VectorSubcoreMesh(
      core_axis_name="c", subcore_axis_name="s", num_cores=NC, num_subcores=NS)

  @functools.partial(
      pl.kernel,
      out_type=jax.ShapeDtypeStruct((NC * n_pad,), jnp.float32),
      mesh=mesh,
      compiler_params=pltpu.CompilerParams(needs_layout_passes=False),
      scratch_types=[
          pltpu.VMEM((rw, 128), jnp.int32),
          pltpu.VMEM((rw, 128), jnp.float32),
          pltpu.VMEM((640,), jnp.float32),
          pltpu.VMEM_SHARED((n_pad,), jnp.float32),
          pltpu.SemaphoreType.DMA,
      ],
  )
  def deg_k(dst_hbm, w_hbm, out_hbm, dst_v, w_v, zb_v, deg_sh, sem):
    cid = lax.axis_index("c")
    sid = lax.axis_index("s")
    wid = sid * NC + cid

    for i in range(640 // LANE):
      zb_v[pl.ds(i * LANE, LANE)] = jnp.zeros((LANE,), jnp.float32)

    @pl.when(sid == 0)
    def _zero():
      def zloop(j, c):
        pltpu.sync_copy(zb_v.at[pl.ds(0, 640)], deg_sh.at[pl.ds(j * 640, 640)])
        return c
      nfull = n_pad // 640
      lax.fori_loop(0, nfull, zloop, 0)
      rem = n_pad - nfull * 640
      if rem:
        pltpu.sync_copy(zb_v.at[pl.ds(0, rem)], deg_sh.at[pl.ds(nfull * 640, rem)])

    plsc.subcore_barrier()

    pltpu.sync_copy(dst_hbm.at[pl.ds(wid * rw, rw)], dst_v)
    pltpu.sync_copy(w_hbm.at[pl.ds(wid * rw, rw)], w_v)
    k = 8
    for g in range(rw // k):
      ds = []
      for j in range(k):
        r = g * k + j
        ds.append(pltpu.async_copy(
            w_v.at[r], deg_sh.at[dst_v.at[r]], sem, add=True))
      for d in ds:
        d.wait()

    plsc.subcore_barrier()

    @pl.when(sid == 0)
    def _drain():
      pltpu.sync_copy(deg_sh, out_hbm.at[pl.ds(cid * n_pad, n_pad)])

  return deg_k


def _tcfuse_kernel(t_steps: int, n_nodes: int, c_dim: int, nb: int):
  bn = n_nodes // nb

  def body(xm1, x0, xp1, cw3, gw, cb, degp, y_ref, dinv_ref):
    g0 = jnp.dot(cw3[0], gw[...], preferred_element_type=jnp.float32)
    g1 = jnp.dot(cw3[1], gw[...], preferred_element_type=jnp.float32)
    g2 = jnp.dot(cw3[2], gw[...], preferred_element_type=jnp.float32)
    acc = jnp.dot(xm1[0], g0, preferred_element_type=jnp.float32)
    acc = acc + jnp.dot(x0[0], g1, preferred_element_type=jnp.float32)
    acc = acc + jnp.dot(xp1[0], g2, preferred_element_type=jnp.float32)
    acc = acc + jnp.dot(cb[...], gw[...], preferred_element_type=jnp.float32)
    deg = degp[0] + degp[1] + 1.0
    dinv = lax.rsqrt(deg)
    y_ref[0] = acc * dinv
    dinv_ref[...] = dinv

  return pl.pallas_call(
      body,
      grid=(t_steps, nb),
      in_specs=[
          pl.BlockSpec((1, bn, c_dim), lambda t, n: (t, n, 0)),
          pl.BlockSpec((1, bn, c_dim), lambda t, n: (t + 1, n, 0)),
          pl.BlockSpec((1, bn, c_dim), lambda t, n: (t + 2, n, 0)),
          pl.BlockSpec((3, c_dim, c_dim), lambda t, n: (0, 0, 0)),
          pl.BlockSpec((c_dim, c_dim), lambda t, n: (0, 0)),
          pl.BlockSpec((1, c_dim), lambda t, n: (0, 0)),
          pl.BlockSpec((NC, bn, 1), lambda t, n: (0, n, 0)),
      ],
      out_specs=[
          pl.BlockSpec((1, bn, c_dim), lambda t, n: (t, n, 0)),
          pl.BlockSpec((bn, 1), lambda t, n: (n, 0)),
      ],
      out_shape=[
          jax.ShapeDtypeStruct((t_steps, n_nodes, c_dim), jnp.float32),
          jax.ShapeDtypeStruct((n_nodes, 1), jnp.float32),
      ],
  )


def _agg_kernel(t_steps: int, n_nodes: int, c_dim: int, n_rows: int,
                n_pad: int):
  rw = n_rows // NW
  crows = rw // 2
  nchunks = rw // crows
  stripe = n_pad // NS
  BE = 128
  drows = n_pad // 128

  mesh = plsc.VectorSubcoreMesh(
      core_axis_name="c", subcore_axis_name="s", num_cores=NC, num_subcores=NS)

  @functools.partial(
      pl.kernel,
      out_type=[
          jax.ShapeDtypeStruct((t_steps * NC * n_pad, c_dim), jnp.float32),
          jax.ShapeDtypeStruct((n_rows, 128), jnp.float32),
      ],
      mesh=mesh,
      compiler_params=pltpu.CompilerParams(needs_layout_passes=False),
      scratch_types=[
          pltpu.VMEM((crows, 128), jnp.int32),
          pltpu.VMEM((crows, 128), jnp.int32),
          pltpu.VMEM((crows, 128), jnp.float32),
          pltpu.VMEM((BE, c_dim), jnp.float32),
          pltpu.VMEM((BE, c_dim), jnp.float32),
          pltpu.VMEM_SHARED((n_pad, c_dim), jnp.float32),
          pltpu.SemaphoreType.DMA,
          pltpu.SemaphoreType.DMA,
          pltpu.SemaphoreType.DMA,
          pltpu.SemaphoreType.DMA,
      ],
  )
  def agg_k(y_hbm, src_hbm, dst_hbm, w_hbm, dinv_hbm, part_hbm, z_hbm,
            src_v, dst_v, z_v, rows_a, rows_b, h_sh,
            semg0, semg1, sems0, sems1):
    cid = lax.axis_index("c")
    sid = lax.axis_index("s")
    wid = sid * NC + cid
    bufs = (rows_a, rows_b)
    semg = (semg0, semg1)
    sems = (sems0, sems1)

    pltpu.sync_copy(dinv_hbm, rows_a.at[pl.ds(0, drows)])
    def zchunk(ck, c):
      row0 = wid * rw + ck * crows
      pltpu.sync_copy(dst_hbm.at[pl.ds(row0, crows)], dst_v)
      pltpu.sync_copy(w_hbm.at[pl.ds(row0, crows)], rows_b.at[pl.ds(0, crows)])
      def zr(j, cc):
        for i in range(128 // LANE):
          sl = pl.ds(i * LANE, LANE)
          d = dst_v[j, sl]
          dv = plsc.load_gather(
              rows_a, [lax.shift_right_logical(d, 7),
                       lax.bitwise_and(d, 127)])
          z_v[j, sl] = rows_b[j, sl] * dv
        return cc
      lax.fori_loop(0, crows, zr, 0)
      pltpu.sync_copy(z_v, z_hbm.at[pl.ds(row0, crows)])
      return c
    lax.fori_loop(0, nchunks, zchunk, 0)

    zv = jnp.zeros((LANE,), jnp.float32)
    nb = crows

    def tloop(t, carry):
      def zrow(r, c):
        for j in range(c_dim // LANE):
          rows_a[r, pl.ds(j * LANE, LANE)] = zv
        return c
      lax.fori_loop(0, BE, zrow, 0)
      base = sid * stripe
      nfull = stripe // BE
      for q in range(nfull):
        pltpu.sync_copy(rows_a, h_sh.at[pl.ds(base + q * BE, BE)])
      rem = stripe - nfull * BE
      if rem:
        pltpu.sync_copy(rows_a.at[pl.ds(0, rem)],
                        h_sh.at[pl.ds(base + nfull * BE, rem)])
      plsc.subcore_barrier()

      tN = t * n_nodes

      def chunk(ck, c):
        row0 = wid * rw + ck * crows
        pltpu.sync_copy(src_hbm.at[pl.ds(row0, crows)], src_v)
        pltpu.sync_copy(dst_hbm.at[pl.ds(row0, crows)], dst_v)
        pltpu.sync_copy(z_hbm.at[pl.ds(row0, crows)], z_v)
        def gix(j, cc):
          for i in range(128 // LANE):
            sl = pl.ds(i * LANE, LANE)
            src_v[j, sl] = src_v[j, sl] + tN
          return cc
        lax.fori_loop(0, crows, gix, 0)

        pltpu.async_copy(y_hbm.at[src_v.at[0]], bufs[0], semg[0])

        def do_block(b, p):
          buf, sg, ss = bufs[p], semg[p], sems[p]
          obuf, osg, oss = bufs[1 - p], semg[1 - p], sems[1 - p]

          @pl.when(b > 0)
          def _wait_other_scatter():
            pltpu.make_async_copy(obuf, h_sh.at[dst_v.at[b]], oss).wait()

          @pl.when(b + 1 < nb)
          def _prefetch():
            pltpu.async_copy(y_hbm.at[src_v.at[b + 1]], obuf, osg)

          pltpu.make_async_copy(y_hbm.at[src_v.at[b]], buf, sg).wait()

          zrow_ref = z_v.at[b]
          @plsc.parallel_loop(0, BE, unroll=4)
          def scale(e):
            zb = plsc.load_gather(zrow_ref, [jnp.full((LANE,), e, jnp.int32)])
            for j in range(c_dim // LANE):
              sl = pl.ds(j * LANE, LANE)
              buf[e, sl] = buf[e, sl] * zb

          pltpu.async_copy(buf, h_sh.at[dst_v.at[b]], ss, add=True)

        def blk(b, cc):
          @pl.when(b % 2 == 0)
          def _even():
            do_block(b, 0)
          @pl.when(b % 2 == 1)
          def _odd():
            do_block(b, 1)
          return cc
        lax.fori_loop(0, nb, blk, 0)

        lp = (nb - 1) % 2
        pltpu.make_async_copy(
            bufs[lp], h_sh.at[dst_v.at[0]], sems[lp]).wait()
        return c
      lax.fori_loop(0, nchunks, chunk, 0)
      plsc.subcore_barrier()

      ob = (t * NC + cid) * n_pad + sid * stripe
      pltpu.sync_copy(h_sh.at[pl.ds(sid * stripe, stripe)],
                      part_hbm.at[pl.ds(ob, stripe)])
      return carry
    lax.fori_loop(0, t_steps, tloop, 0)

  return agg_k


def _bn_kernel(t_steps: int, n_nodes: int, c_dim: int, n_pad: int):
  def body(part, y, dinv, gb, gamma, beta, out_ref):
    h = (part[0, 0, :n_nodes] + part[0, 1, :n_nodes]
         + y[0] * dinv[...] + gb[...])
    mu = jnp.mean(h, axis=0, keepdims=True)
    var = jnp.mean((h - mu) * (h - mu), axis=0, keepdims=True)
    hn = (h - mu) * lax.rsqrt(var + 1e-5) * gamma[...] + beta[...]
    out_ref[0] = jnp.maximum(hn, 0.0)

  return pl.pallas_call(
      body,
      grid=(t_steps,),
      in_specs=[
          pl.BlockSpec((1, NC, n_pad, c_dim), lambda t: (t, 0, 0, 0)),
          pl.BlockSpec((1, n_nodes, c_dim), lambda t: (t, 0, 0)),
          pl.BlockSpec((n_nodes, 1), lambda t: (0, 0)),
          pl.BlockSpec((1, c_dim), lambda t: (0, 0)),
          pl.BlockSpec((1, c_dim), lambda t: (0, 0)),
          pl.BlockSpec((1, c_dim), lambda t: (0, 0)),
      ],
      out_specs=pl.BlockSpec((1, n_nodes, c_dim), lambda t: (t, 0, 0)),
      out_shape=jax.ShapeDtypeStruct((t_steps, n_nodes, c_dim), jnp.float32),
  )


def kernel(x, edge_index, edge_weight, conv_w, conv_b, gcn_w, gcn_b,
           bn_gamma, bn_beta):
  t_steps, n_nodes, c_dim = x.shape
  e_edges = edge_weight.shape[0]

  ce = 256
  per_w = -(-e_edges // (NW * ce)) * ce
  e2 = per_w * NW
  pad = e2 - e_edges
  fill = (jnp.arange(pad, dtype=jnp.int32) % n_nodes)
  src = jnp.concatenate([edge_index[0].astype(jnp.int32), fill])
  dst = jnp.concatenate([edge_index[1].astype(jnp.int32), fill])
  w = jnp.concatenate([edge_weight, jnp.zeros((pad,), jnp.float32)])
  n_rows = e2 // 128
  src2d = src.reshape(n_rows, 128)
  dst2d = dst.reshape(n_rows, 128)
  w2d = w.reshape(n_rows, 128)

  n_pad = -(-n_nodes // (NS * 8)) * (NS * 8)

  degp = _deg_kernel(n_nodes, n_rows, n_pad)(dst2d, w2d)
  degp = degp.reshape(NC, n_pad)[:, :n_nodes]

  xpad = jnp.concatenate(
      [jnp.zeros((1, n_nodes, c_dim), jnp.float32), x,
       jnp.zeros((1, n_nodes, c_dim), jnp.float32)], axis=0)
  cw3 = conv_w.transpose(2, 1, 0)
  y, dinv2d = _tcfuse_kernel(t_steps, n_nodes, c_dim, 5)(
      xpad, xpad, xpad, cw3, gcn_w, conv_b.reshape(1, c_dim),
      degp.reshape(NC, n_nodes, 1))

  yflat = y.reshape(t_steps * n_nodes, c_dim)
  dinv1 = jnp.concatenate(
      [dinv2d.reshape(n_nodes), jnp.zeros((n_pad - n_nodes,), jnp.float32)])
  part, _ = _agg_kernel(t_steps, n_nodes, c_dim, n_rows, n_pad)(
      yflat, src2d, dst2d, w2d, dinv1.reshape(n_pad // 128, 128))

  out = _bn_kernel(t_steps, n_nodes, c_dim, n_pad)(
      part.reshape(t_steps, NC, n_pad, c_dim), y, dinv2d,
      gcn_b.reshape(1, c_dim), bn_gamma.reshape(1, c_dim),
      bn_beta.reshape(1, c_dim))
  return out
